# baseline (speedup 1.0000x reference)
import functools
import jax
import jax.numpy as jnp
from jax import lax
from jax.experimental import pallas as pl
from jax.experimental.pallas import tpu as pltpu

F32 = jnp.float32
BF16 = jnp.bfloat16

B = 4
SEQ = 2048
D = 2048
H = 16
DH = 128
L = 64
N = B * L
E = 8
TOPK = 2
FF = 4 * D


def _mm_body(x_ref, w_ref, b_ref, o_ref, acc_ref, *, nk, act, out_dtype):
    @pl.when(pl.program_id(2) == 0)
    def _():
        acc_ref[...] = jnp.zeros_like(acc_ref)

    acc_ref[...] += jnp.dot(x_ref[...], w_ref[...], preferred_element_type=F32)

    @pl.when(pl.program_id(2) == nk - 1)
    def _():
        r = acc_ref[...]
        if b_ref is not None:
            r = r + b_ref[...]
        if act == 'gelu':
            r = jax.nn.gelu(r)
        o_ref[...] = r.astype(out_dtype)


def _mm(x, w, bias=None, act=None, bm=256, bn=1024, bk=2048, out_dtype=F32):
    x = x.astype(BF16)
    w = w.astype(BF16)
    M, K = x.shape
    Nn = w.shape[1]
    nm, nn, nk = M // bm, Nn // bn, K // bk
    in_specs = [
        pl.BlockSpec((bm, bk), lambda i, j, k: (i, k)),
        pl.BlockSpec((bk, bn), lambda i, j, k: (k, j)),
    ]
    args = [x, w]
    if bias is not None:
        args.append(bias.reshape(1, Nn).astype(F32))
        in_specs.append(pl.BlockSpec((1, bn), lambda i, j, k: (0, j)))
        body = lambda xr, wr, br, orf, ar: _mm_body(
            xr, wr, br, orf, ar, nk=nk, act=act, out_dtype=out_dtype)
    else:
        body = lambda xr, wr, orf, ar: _mm_body(
            xr, wr, None, orf, ar, nk=nk, act=act, out_dtype=out_dtype)
    return pl.pallas_call(
        body,
        grid=(nm, nn, nk),
        in_specs=in_specs,
        out_specs=pl.BlockSpec((bm, bn), lambda i, j, k: (i, j)),
        out_shape=jax.ShapeDtypeStruct((M, Nn), out_dtype),
        scratch_shapes=[pltpu.VMEM((bm, bn), F32)],
        compiler_params=pltpu.CompilerParams(
            dimension_semantics=("parallel", "parallel", "arbitrary")),
    )(*args)


def _attn_body(q_ref, k_ref, v_ref, kl_ref, vl_ref, o_ref):
    q = q_ref[...]
    scale = jnp.sqrt(jnp.float32(DH))
    dn = (((1,), (1,)), ((), ()))
    s1 = lax.dot_general(q, k_ref[...], dn,
                         preferred_element_type=F32) / scale
    s2 = lax.dot_general(q, kl_ref[...], dn,
                         preferred_element_type=F32) / scale
    m = jnp.maximum(jnp.max(s1, axis=1, keepdims=True),
                    jnp.max(s2, axis=1, keepdims=True))
    e1 = jnp.exp(s1 - m)
    e2 = jnp.exp(s2 - m)
    den = jnp.sum(e1, axis=1, keepdims=True) + jnp.sum(e2, axis=1, keepdims=True)
    p1 = (e1 / den).astype(BF16)
    p2 = (e2 / den).astype(BF16)
    o_ref[...] = (jnp.dot(p1, v_ref[...], preferred_element_type=F32)
                  + jnp.dot(p2, vl_ref[...], preferred_element_type=F32))


def _attn_core(q2d, k2d, v2d, kl2d, vl2d):
    return pl.pallas_call(
        _attn_body,
        grid=(B, H),
        in_specs=[
            pl.BlockSpec((L, DH), lambda b, h: (b, h)),
            pl.BlockSpec((SEQ, DH), lambda b, h: (b, h)),
            pl.BlockSpec((SEQ, DH), lambda b, h: (b, h)),
            pl.BlockSpec((L, DH), lambda b, h: (b, h)),
            pl.BlockSpec((L, DH), lambda b, h: (b, h)),
        ],
        out_specs=pl.BlockSpec((L, DH), lambda b, h: (b, h)),
        out_shape=jax.ShapeDtypeStruct((N, D), F32),
        compiler_params=pltpu.CompilerParams(
            dimension_semantics=("parallel", "arbitrary")),
    )(q2d, k2d, v2d, kl2d, vl2d)


def _experts_body(lat_ref, fc1_ref, b1_ref, fc2_ref, b2_ref, w_ref, o_ref):
    e = pl.program_id(0)
    fb = pl.program_id(1)

    @pl.when((e == 0) & (fb == 0))
    def _():
        o_ref[...] = jnp.zeros_like(o_ref)

    h = jnp.dot(lat_ref[...], fc1_ref[0], preferred_element_type=F32)
    h = jax.nn.gelu(h + b1_ref[0])
    part = jnp.dot(h.astype(BF16), fc2_ref[0], preferred_element_type=F32)
    w = w_ref[0][:, 0:1]

    @pl.when(fb == 0)
    def _():
        o_ref[...] += w * b2_ref[0]

    o_ref[...] += w * part


def _experts(lat_bf, fc1, b1, fc2, b2, w_full):
    bf = 2048
    nf = FF // bf
    wmat = jnp.broadcast_to(w_full.T[:, :, None], (E, N, 128)).astype(F32)
    return pl.pallas_call(
        _experts_body,
        grid=(E, nf),
        in_specs=[
            pl.BlockSpec((N, D), lambda e, f: (0, 0)),
            pl.BlockSpec((1, D, bf), lambda e, f: (e, 0, f)),
            pl.BlockSpec((1, 1, bf), lambda e, f: (e, 0, f)),
            pl.BlockSpec((1, bf, D), lambda e, f: (e, f, 0)),
            pl.BlockSpec((1, 1, D), lambda e, f: (e, 0, 0)),
            pl.BlockSpec((1, N, 128), lambda e, f: (e, 0, 0)),
        ],
        out_specs=pl.BlockSpec((N, D), lambda e, f: (0, 0)),
        out_shape=jax.ShapeDtypeStruct((N, D), F32),
        compiler_params=pltpu.CompilerParams(
            dimension_semantics=("arbitrary", "arbitrary")),
    )(lat_bf.astype(BF16), fc1.astype(BF16), b1.reshape(E, 1, FF),
      fc2.astype(BF16), b2.reshape(E, 1, D), wmat)


def _layer(lat2d, src_bf, q_W, k_W, v_W, o_W, ff1_W, ff1_b, ff2_W, ff2_b):
    lat_bf = lat2d.astype(BF16)
    q2d = _mm(lat_bf, q_W, bm=256, bn=1024, out_dtype=BF16)
    k2d = _mm(src_bf, k_W, bm=512, bn=1024, out_dtype=BF16)
    v2d = _mm(src_bf, v_W, bm=512, bn=1024, out_dtype=BF16)
    kl2d = _mm(lat_bf, k_W, bm=256, bn=1024, out_dtype=BF16)
    vl2d = _mm(lat_bf, v_W, bm=256, bn=1024, out_dtype=BF16)
    ctx = _attn_core(q2d, k2d, v2d, kl2d, vl2d)
    lat2d = lat2d + _mm(ctx, o_W, bm=256, bn=1024)
    ffh = _mm(lat2d, ff1_W, ff1_b, act='gelu', bm=256, bn=2048,
              out_dtype=BF16)
    lat2d = lat2d + _mm(ffh, ff2_W, ff2_b, bm=256, bn=2048, bk=2048)
    return lat2d


def kernel(src_hidden, latents_init, in_proj_W, in_proj_b,
           layer0_q_W, layer0_k_W, layer0_v_W, layer0_o_W,
           layer0_ff1_W, layer0_ff1_b, layer0_ff2_W, layer0_ff2_b,
           layer1_q_W, layer1_k_W, layer1_v_W, layer1_o_W,
           layer1_ff1_W, layer1_ff1_b, layer1_ff2_W, layer1_ff2_b,
           router_W, exp_fc1_W, exp_fc1_b, exp_fc2_W, exp_fc2_b,
           sh_fc1_W, sh_fc1_b, sh_fc2_W, sh_fc2_b,
           shared_expert_weight, output_scale):
    x = src_hidden.reshape(B * SEQ, D)
    src_bf = _mm(x, in_proj_W, in_proj_b, bm=512, bn=1024, out_dtype=BF16)

    lat2d = jnp.broadcast_to(latents_init[None], (B, L, D)).reshape(N, D)
    lat2d = _layer(lat2d, src_bf, layer0_q_W, layer0_k_W, layer0_v_W,
                   layer0_o_W, layer0_ff1_W, layer0_ff1_b, layer0_ff2_W,
                   layer0_ff2_b)
    lat2d = _layer(lat2d, src_bf, layer1_q_W, layer1_k_W, layer1_v_W,
                   layer1_o_W, layer1_ff1_W, layer1_ff1_b, layer1_ff2_W,
                   layer1_ff2_b)

    rw = jnp.pad(router_W, ((0, 0), (0, 128 - E)))
    logits = _mm(lat2d, rw, bm=256, bn=128)[:, :E]
    probs = jax.nn.softmax(logits, axis=-1)
    topw, topi = lax.top_k(probs, TOPK)
    topw = topw / (jnp.sum(topw, axis=-1, keepdims=True) + 1e-08)
    one_hot = jax.nn.one_hot(topi, E)
    w_full = jnp.sum(one_hot * topw[..., None], axis=1)

    eout = _experts(lat2d, exp_fc1_W, exp_fc1_b, exp_fc2_W, exp_fc2_b, w_full)
    shh = _mm(lat2d, sh_fc1_W, sh_fc1_b, act='gelu', bm=256, bn=1024,
              out_dtype=BF16)
    sh = _mm(shh, sh_fc2_W, sh_fc2_b, bm=256, bn=1024)

    expert_output = eout + shared_expert_weight * sh
    soft = lat2d + expert_output
    rms = jnp.sqrt(jnp.mean(soft ** 2, axis=-1, keepdims=True) + 1e-08)
    soft = soft / rms * output_scale
    soft_tokens = soft.reshape(B, L, D)

    f = jnp.mean(jnp.sum(one_hot, axis=1), axis=0)
    P = jnp.mean(probs, axis=0)
    aux_loss = 0.01 * E * jnp.sum(f * P)
    routing_entropy = jnp.mean(
        -jnp.sum(probs * jnp.log(probs + 1e-08), axis=-1))
    return soft_tokens, aux_loss, routing_entropy, jnp.var(soft_tokens, ddof=1)
